# baseline (speedup 1.0000x reference)
import functools

import jax
import jax.numpy as jnp
from jax import lax
from jax.experimental import pallas as pl
from jax.experimental.pallas import tpu as pltpu
from jax.experimental.pallas import tpu_sc as plsc

_B, _H, _D = 2048, 512, 32
_NW = 32
_SPW = _B // _NW
_BB = 32
_G = _B // _BB


def _tc_body(targ_ref, idx_ref, coeff_ref, t0_ref):
    blk = targ_ref[...]
    t1 = blk[:, :, 1]
    cnt = jnp.sum((t1 != 0.0).astype(jnp.float32), axis=1)
    ti = cnt.astype(jnp.int32)
    idx = ti - 1
    safe = jnp.maximum(idx, 0)

    h_iota = jax.lax.broadcasted_iota(jnp.int32, (_BB, _H), 1)
    onehot = (h_iota == safe[:, None]).astype(jnp.float32)
    t0 = jnp.sum(blk[:, :, 0] * onehot, axis=1)

    x = cnt * (1.0 / (_H - 1))
    w = 1.0 + 0.7 * (x * x) * jnp.sqrt(x)
    coeff = jnp.where(idx >= 0, w, 0.0) * (_D / _B)

    idx_ref[...] = safe.reshape(1, 1, _BB)
    coeff_ref[...] = coeff.reshape(1, 1, _BB)
    t0_ref[...] = t0.reshape(1, 1, _BB)


def _tc_prepass(targ):
    out3 = lambda dt: jax.ShapeDtypeStruct((_G, 1, _BB), dt)
    spec3 = pl.BlockSpec((1, 1, _BB), lambda i: (i, 0, 0))
    return pl.pallas_call(
        _tc_body,
        grid=(_G,),
        in_specs=[pl.BlockSpec((_BB, _H, _D), lambda i: (i, 0, 0))],
        out_specs=[spec3, spec3, spec3],
        out_shape=[out3(jnp.int32), out3(jnp.float32), out3(jnp.float32)],
    )(targ)


def _sc_body(pred_hbm, idx_hbm, coeff_hbm, t0_hbm, out_hbm,
             idx_v, coeff_v, t0_v, prow, acc_v, psem):
    c = lax.axis_index("c")
    s = lax.axis_index("s")
    wid = s * 2 + c
    base = wid * _SPW

    pltpu.sync_copy(idx_hbm.at[pl.ds(base, _SPW)], idx_v)
    pltpu.sync_copy(coeff_hbm.at[pl.ds(base, _SPW)], coeff_v)
    pltpu.sync_copy(t0_hbm.at[pl.ds(base, _SPW)], t0_v)

    lane = lax.iota(jnp.int32, 16)
    acc = jnp.zeros((16,), jnp.float32)
    for g in range(_SPW // 16):
        idx16 = idx_v[pl.ds(g * 16, 16)]
        pbase16 = (idx16 // 8) * 8
        poff16 = idx16 - pbase16
        handles = []
        for k in range(16):
            handles.append(pltpu.async_copy(
                pred_hbm.at[pl.ds(base + g * 16 + k, 1),
                            pl.ds(pl.multiple_of(pbase16[k], 8), 8), :],
                prow.at[pl.ds(k, 1)], psem))
        for h in handles:
            h.wait()
        p016 = jnp.zeros((16,), jnp.float32)
        lane0 = (lane == 0).astype(jnp.float32)
        for k in range(16):
            chunk = prow[k, poff16[k], pl.ds(0, 16)]
            p0 = jnp.sum(chunk * lane0)
            p016 = jnp.where(lane == k, jnp.full((16,), p0, jnp.float32),
                             p016)
        d = p016 - t0_v[pl.ds(g * 16, 16)]
        acc = acc + coeff_v[pl.ds(g * 16, 16)] * d * d

    acc_v[pl.ds(0, 16)] = acc
    pltpu.sync_copy(acc_v, out_hbm.at[pl.ds(wid * 128, 128)])


def _sc_gather(pred, idx, coeff, t0):
    mesh = plsc.VectorSubcoreMesh(core_axis_name="c", subcore_axis_name="s")
    run = functools.partial(
        pl.kernel,
        mesh=mesh,
        compiler_params=pltpu.CompilerParams(needs_layout_passes=False),
        out_type=jax.ShapeDtypeStruct((_NW * 128,), jnp.float32),
        scratch_types=[
            pltpu.VMEM((_SPW,), jnp.int32),
            pltpu.VMEM((_SPW,), jnp.float32),
            pltpu.VMEM((_SPW,), jnp.float32),
            pltpu.VMEM((16, 8, _D), jnp.float32),
            pltpu.VMEM((128,), jnp.float32),
            pltpu.SemaphoreType.DMA,
        ],
    )(_sc_body)
    return run(pred, idx, coeff, t0)


def kernel(pred, targ, weights):
    idx3, coeff3, t03 = _tc_prepass(targ)
    flat = _sc_gather(pred, idx3.reshape(_B), coeff3.reshape(_B),
                      t03.reshape(_B))
    partials = flat.reshape(_NW, 128)[:, :16]
    loss = jnp.sum(partials)
    return (loss, {"a0_loss": loss})


# speedup vs baseline: 2.1051x; 2.1051x over previous
import functools

import jax
import jax.numpy as jnp
from jax import lax
from jax.experimental import pallas as pl
from jax.experimental.pallas import tpu as pltpu
from jax.experimental.pallas import tpu_sc as plsc

_B, _H, _D = 2048, 512, 32
_HD = _H * _D
_NW = 32
_SPW = _B // _NW
_BB = 64
_G = _B // _BB


def _tc_body(targ_ref, idx_ref, coeff_ref):
    x = targ_ref[...]
    c_iota = jax.lax.broadcasted_iota(jnp.int32, (_BB, _HD), 1)
    is_ch1 = (c_iota % _D == 1).astype(jnp.float32)
    cnt = jnp.sum((x != 0.0).astype(jnp.float32) * is_ch1, axis=1)
    ti = cnt.astype(jnp.int32)
    safe = jnp.maximum(ti - 1, 0)

    xn = cnt * (1.0 / (_H - 1))
    w = 1.0 + 0.7 * (xn * xn) * jnp.sqrt(xn)
    coeff = jnp.where(ti >= 1, w, 0.0) * (_D / _B)

    idx_ref[...] = safe.reshape(1, 1, _BB)
    coeff_ref[...] = coeff.reshape(1, 1, _BB)


def _tc_prepass(targ2):
    out3 = lambda dt: jax.ShapeDtypeStruct((_G, 1, _BB), dt)
    spec3 = pl.BlockSpec((1, 1, _BB), lambda i: (i, 0, 0))
    return pl.pallas_call(
        _tc_body,
        grid=(_G,),
        in_specs=[pl.BlockSpec((_BB, _HD), lambda i: (i, 0))],
        out_specs=[spec3, spec3],
        out_shape=[out3(jnp.int32), out3(jnp.float32)],
    )(targ2)


def _sc_body(pred_hbm, targ_hbm, idx_hbm, coeff_hbm, out_hbm,
             idx_v, coeff_v, pbuf, tbuf, acc_v, psem, tsem):
    c = lax.axis_index("c")
    s = lax.axis_index("s")
    wid = s * 2 + c
    base = wid * _SPW

    pltpu.sync_copy(idx_hbm.at[pl.ds(base, _SPW)], idx_v)
    pltpu.sync_copy(coeff_hbm.at[pl.ds(base, _SPW)], coeff_v)

    lane = lax.iota(jnp.int32, 16)
    lane0 = (lane == 0).astype(jnp.float32)
    acc = jnp.zeros((16,), jnp.float32)

    for g in range(_SPW // 16):
        idx16 = idx_v[pl.ds(g * 16, 16)]
        col16 = idx16 * _D
        c016 = (col16 // 128) * 128
        coff16 = col16 - c016
        handles = []
        for k in range(16):
            j = g * 16 + k
            row8 = base + (j // 8) * 8
            c0k = pl.multiple_of(c016[k], 128)
            handles.append(pltpu.async_copy(
                pred_hbm.at[pl.ds(row8, 8), pl.ds(c0k, 128)],
                pbuf.at[k], psem))
            handles.append(pltpu.async_copy(
                targ_hbm.at[pl.ds(row8, 8), pl.ds(c0k, 128)],
                tbuf.at[k], tsem))
        for h in handles:
            h.wait()
        p016 = jnp.zeros((16,), jnp.float32)
        t016 = jnp.zeros((16,), jnp.float32)
        for k in range(16):
            j = g * 16 + k
            co = pl.multiple_of(coff16[k], 32)
            pchunk = pbuf[k, j % 8, pl.ds(co, 16)]
            tchunk = tbuf[k, j % 8, pl.ds(co, 16)]
            p0 = jnp.sum(pchunk * lane0)
            t0 = jnp.sum(tchunk * lane0)
            sel = lane == k
            p016 = jnp.where(sel, jnp.full((16,), p0, jnp.float32), p016)
            t016 = jnp.where(sel, jnp.full((16,), t0, jnp.float32), t016)
        d = p016 - t016
        acc = acc + coeff_v[pl.ds(g * 16, 16)] * d * d

    acc_v[pl.ds(0, 16)] = acc
    pltpu.sync_copy(acc_v, out_hbm.at[pl.ds(wid * 128, 128)])


def _sc_gather(pred2, targ2, idx, coeff):
    mesh = plsc.VectorSubcoreMesh(core_axis_name="c", subcore_axis_name="s")
    run = functools.partial(
        pl.kernel,
        mesh=mesh,
        compiler_params=pltpu.CompilerParams(needs_layout_passes=False),
        out_type=jax.ShapeDtypeStruct((_NW * 128,), jnp.float32),
        scratch_types=[
            pltpu.VMEM((_SPW,), jnp.int32),
            pltpu.VMEM((_SPW,), jnp.float32),
            pltpu.VMEM((16, 8, 128), jnp.float32),
            pltpu.VMEM((16, 8, 128), jnp.float32),
            pltpu.VMEM((128,), jnp.float32),
            pltpu.SemaphoreType.DMA,
            pltpu.SemaphoreType.DMA,
        ],
    )(_sc_body)
    return run(pred2, targ2, idx, coeff)


def kernel(pred, targ, weights):
    pred2 = pred.reshape(_B, _HD)
    targ2 = targ.reshape(_B, _HD)
    idx3, coeff3 = _tc_prepass(targ2)
    flat = _sc_gather(pred2, targ2, idx3.reshape(_B), coeff3.reshape(_B))
    partials = flat.reshape(_NW, 128)[:, :16]
    loss = jnp.sum(partials)
    return (loss, {"a0_loss": loss})


# speedup vs baseline: 18.4815x; 8.7794x over previous
import functools

import numpy as np
import jax
import jax.numpy as jnp
from jax import lax
from jax.experimental import pallas as pl
from jax.experimental.pallas import tpu as pltpu
from jax.experimental.pallas import tpu_sc as plsc

_B, _H, _D = 2048, 512, 32
_NW = 32
_SPW = _B // _NW
_NT = 4
_LUT = 1024


def _w_table():
    t = np.arange(_LUT, dtype=np.float64)
    t = np.minimum(t, _H)
    w = 1.0 + 0.7 * (t / (_H - 1)) ** 2.5
    w = w * (_D / _B)
    return jnp.asarray(w.astype(np.float32))


def _sc_body(pred_hbm, targ_hbm, lut_hbm, out_hbm,
             lut_v, tw, pw, acc_v, tsems, psem):
    c = lax.axis_index("c")
    s = lax.axis_index("s")
    wid = s * 2 + c
    base = wid * _SPW

    pltpu.sync_copy(lut_hbm, lut_v)

    lane = lax.iota(jnp.int32, 16)
    lane0 = (lane == 0).astype(jnp.float32)
    acc = jnp.zeros((16,), jnp.float32)

    def stage_targ(j):
        return pltpu.async_copy(
            targ_hbm.at[base + j, pl.ds(0, 8), :], tw.at[j % _NT],
            tsems.at[j % _NT])

    tpend = [stage_targ(j) for j in range(_NT)]

    for g in range(_SPW // 16):
        coeffv = jnp.zeros((16,), jnp.float32)
        t0v = jnp.zeros((16,), jnp.float32)
        pred_handles = []
        offs = []
        for k in range(16):
            j = g * 16 + k
            b = j % _NT
            tpend[b].wait()

            def cbody(ci, cnt, b=b):
                vals = tw[b, 1, pl.ds(pl.multiple_of(ci * 16, 16), 16)]
                return cnt + (vals != 0.0).astype(jnp.float32)

            cnt = lax.fori_loop(0, _H // 16, cbody,
                                jnp.zeros((16,), jnp.float32))
            t = jnp.sum(cnt)
            ti = t.astype(jnp.int32)
            safe = jnp.maximum(ti - 1, 0)

            sub = (lane == safe % 16).astype(jnp.float32)
            co16 = pl.multiple_of((safe // 16) * 16, 16)
            t0 = jnp.sum(tw[b, 0, pl.ds(co16, 16)] * sub)

            lsel = (lane == ti % 16).astype(jnp.float32)
            lo16 = pl.multiple_of((ti // 16) * 16, 16)
            coeff = jnp.sum(lut_v[pl.ds(lo16, 16)] * lsel)
            coeff = jnp.where(ti >= 1, coeff, 0.0)

            cb = pl.multiple_of((safe // 128) * 128, 128)
            offs.append((safe % 128 // 16) * 16)
            pred_handles.append(pltpu.async_copy(
                pred_hbm.at[base + j, pl.ds(0, 8), pl.ds(cb, 128)],
                pw.at[k], psem))

            sel = lane == k
            t0v = jnp.where(sel, jnp.full((16,), t0, jnp.float32), t0v)
            coeffv = jnp.where(sel, jnp.full((16,), coeff, jnp.float32),
                               coeffv)
            offs[-1] = (offs[-1], sub)

            if j + _NT < _SPW:
                tpend[b] = stage_targ(j + _NT)

        for h in pred_handles:
            h.wait()
        p0v = jnp.zeros((16,), jnp.float32)
        for k in range(16):
            po, sub = offs[k]
            chunk = pw[k, 0, pl.ds(pl.multiple_of(po, 16), 16)]
            p0 = jnp.sum(chunk * sub)
            p0v = jnp.where(lane == k, jnp.full((16,), p0, jnp.float32), p0v)
        d = p0v - t0v
        acc = acc + coeffv * d * d

    acc_v[pl.ds(0, 16)] = acc
    pltpu.sync_copy(acc_v, out_hbm.at[pl.ds(wid * 128, 128)])


def kernel(pred, targ, weights):
    predT = jnp.transpose(pred, (0, 2, 1))
    targT = jnp.transpose(targ, (0, 2, 1))
    lut = _w_table()

    mesh = plsc.VectorSubcoreMesh(core_axis_name="c", subcore_axis_name="s")
    run = functools.partial(
        pl.kernel,
        mesh=mesh,
        compiler_params=pltpu.CompilerParams(needs_layout_passes=False),
        out_type=jax.ShapeDtypeStruct((_NW * 128,), jnp.float32),
        scratch_types=[
            pltpu.VMEM((_LUT,), jnp.float32),
            pltpu.VMEM((_NT, 8, _H), jnp.float32),
            pltpu.VMEM((16, 8, 128), jnp.float32),
            pltpu.VMEM((128,), jnp.float32),
            pltpu.SemaphoreType.DMA((_NT,)),
            pltpu.SemaphoreType.DMA,
        ],
    )(_sc_body)

    flat = run(predT, targT, lut)
    partials = flat.reshape(_NW, 128)[:, :16]
    loss = jnp.sum(partials)
    return (loss, {"a0_loss": loss})
